# baseline (speedup 1.0000x reference)
import jax
import jax.numpy as jnp
from jax import lax
from jax.experimental import pallas as pl
from jax.experimental.pallas import tpu as pltpu
from jax.experimental.pallas import tpu_sc as plsc

VOCAB = 1000000
EMBED = 64
BATCH = 16384
NUM_NEG = 5
NSCORE = 1 + NUM_NEG
LANES = 16

NC = 2
NS = 16
NW = NC * NS
BPW = BATCH // NW
CHUNK = 64
NCHUNK = BPW // CHUNK
PW = NSCORE * LANES

TW = 512


def _tr_body(x_ref, o_ref):
    o_ref[...] = x_ref[...].T


def _transpose(xt):
    nblk = (VOCAB + TW - 1) // TW
    return pl.pallas_call(
        _tr_body,
        grid=(nblk,),
        in_specs=[pl.BlockSpec((EMBED, TW), lambda i: (0, i))],
        out_specs=pl.BlockSpec((TW, EMBED), lambda i: (i, 0)),
        out_shape=jax.ShapeDtypeStruct((VOCAB, EMBED), jnp.float32),
    )(xt)


def _sc_scores_body(cidx_hbm, xidx_hbm, nidx_hbm, ev_hbm, eu_hbm, out_hbm,
                    cidx, xidx, nidx, vrows, urows, nrows, scores, sem):
    c = lax.axis_index("c")
    s = lax.axis_index("s")
    wid = s * NC + c
    base = wid * BPW

    pltpu.sync_copy(cidx_hbm.at[pl.ds(base, BPW)], cidx)
    pltpu.sync_copy(xidx_hbm.at[pl.ds(base, BPW)], xidx)
    pltpu.sync_copy(nidx_hbm.at[pl.ds(base * NUM_NEG, BPW * NUM_NEG)], nidx)

    for j in range(NCHUNK):
        def issue(g, carry, j=j):
            cv = cidx[pl.ds(j * CHUNK + g * 16, 16)]
            xv = xidx[pl.ds(j * CHUNK + g * 16, 16)]
            nv = [nidx[pl.ds((j * CHUNK) * NUM_NEG + g * 80 + 16 * i, 16)]
                  for i in range(NUM_NEG)]
            for l in range(16):
                e = g * 16 + l
                pltpu.async_copy(ev_hbm.at[cv[l]], vrows.at[e], sem)
                pltpu.async_copy(eu_hbm.at[xv[l]], urows.at[e], sem)
                for k in range(NUM_NEG):
                    p = l * NUM_NEG + k
                    pltpu.async_copy(eu_hbm.at[nv[p // 16][p % 16]],
                                     nrows.at[e * NUM_NEG + k], sem)
            return carry

        lax.fori_loop(0, CHUNK // 16, issue, 0)

        def drain(e, carry):
            for _ in range(1 + 1 + NUM_NEG):
                pltpu.make_async_copy(ev_hbm.at[0], vrows.at[0], sem).wait()
            return carry

        lax.fori_loop(0, CHUNK, drain, 0)

        def elem(e, carry, j=j):
            u = [urows[e, pl.ds(16 * t, 16)] for t in range(4)]
            off = (j * CHUNK + e) * PW
            accp = vrows[e, pl.ds(0, 16)] * u[0]
            for t in range(1, 4):
                accp += vrows[e, pl.ds(16 * t, 16)] * u[t]
            scores[pl.ds(off, 16)] = accp
            for k in range(NUM_NEG):
                r = e * NUM_NEG + k
                accn = nrows[r, pl.ds(0, 16)] * u[0]
                for t in range(1, 4):
                    accn += nrows[r, pl.ds(16 * t, 16)] * u[t]
                scores[pl.ds(off + (1 + k) * 16, 16)] = -accn
            return carry

        lax.fori_loop(0, CHUNK, elem, 0)

    pltpu.sync_copy(scores, out_hbm.at[pl.ds(base * PW, BPW * PW)])


def _sc_scores(center, context, negflat, embed_v, embed_u):
    mesh = plsc.VectorSubcoreMesh(core_axis_name="c", subcore_axis_name="s")
    f = pl.kernel(
        _sc_scores_body,
        out_type=jax.ShapeDtypeStruct((BATCH * PW,), jnp.float32),
        mesh=mesh,
        scratch_types=[
            pltpu.VMEM((BPW,), jnp.int32),
            pltpu.VMEM((BPW,), jnp.int32),
            pltpu.VMEM((BPW * NUM_NEG,), jnp.int32),
            pltpu.VMEM((CHUNK, EMBED), jnp.float32),
            pltpu.VMEM((CHUNK, EMBED), jnp.float32),
            pltpu.VMEM((CHUNK * NUM_NEG, EMBED), jnp.float32),
            pltpu.VMEM((BPW * PW,), jnp.float32),
            pltpu.SemaphoreType.DMA,
        ],
    )
    return f(center, context, negflat, embed_v, embed_u)


def _loss_body(p_ref, o_ref):
    x = p_ref[...]
    g = lax.broadcasted_iota(jnp.int32, (128, 8), 0) // 16
    t = lax.broadcasted_iota(jnp.int32, (128, 8), 1)
    m = jnp.where(g == t, 1.0, 0.0).astype(jnp.float32)
    s = jax.lax.dot_general(x, m, (((1,), (0,)), ((), ())),
                            preferred_element_type=jnp.float32)
    o_ref[0, 0] = jnp.sum(jnp.log(jax.nn.sigmoid(s))) * (-1.0 / BATCH)


def kernel(center_word, context_word, negative_samples, embed_v, embed_u):
    center = center_word.astype(jnp.int32)
    context = context_word.astype(jnp.int32)
    negflat = negative_samples.astype(jnp.int32).reshape(-1)
    ev_rm = _transpose(embed_v.T)
    eu_rm = _transpose(embed_u.T)
    partials = _sc_scores(center, context, negflat, ev_rm, eu_rm)
    partials2d = partials.reshape(BATCH * PW // 128, 128)
    loss = pl.pallas_call(
        _loss_body,
        out_shape=jax.ShapeDtypeStruct((1, 1), jnp.float32),
        out_specs=pl.BlockSpec(memory_space=pltpu.SMEM),
    )(partials2d)
    return loss[0, 0]


# speedup vs baseline: 3.3689x; 3.3689x over previous
import jax
import jax.numpy as jnp
from jax import lax
from jax.experimental import pallas as pl
from jax.experimental.pallas import tpu as pltpu
from jax.experimental.pallas import tpu_sc as plsc

VOCAB = 1000000
EMBED = 64
BATCH = 16384
NUM_NEG = 5
LANES = 16

NC = 2
NS = 16
NW = NC * NS
BPW = BATCH // NW
CHUNK = 64
NCHUNK = BPW // CHUNK
PWN = NUM_NEG * LANES


def _sc_u_body(xidx_hbm, nidx_hbm, eu_hbm, out_hbm, urows_hbm,
               xidx, nidx, urows, nrows, partials, sem):
    c = lax.axis_index("c")
    s = lax.axis_index("s")
    wid = s * NC + c
    base = wid * BPW

    pltpu.sync_copy(xidx_hbm.at[pl.ds(base, BPW)], xidx)
    pltpu.sync_copy(nidx_hbm.at[pl.ds(base * NUM_NEG, BPW * NUM_NEG)], nidx)

    for j in range(NCHUNK):
        def issue(g, carry, j=j):
            xv = xidx[pl.ds(j * CHUNK + g * 16, 16)]
            nv = [nidx[pl.ds((j * CHUNK) * NUM_NEG + g * 80 + 16 * i, 16)]
                  for i in range(NUM_NEG)]
            for l in range(16):
                e = g * 16 + l
                pltpu.async_copy(eu_hbm.at[xv[l]], urows.at[e], sem)
                for k in range(NUM_NEG):
                    p = l * NUM_NEG + k
                    pltpu.async_copy(eu_hbm.at[nv[p // 16][p % 16]],
                                     nrows.at[e * NUM_NEG + k], sem)
            return carry

        lax.fori_loop(0, CHUNK // 16, issue, 0)

        def drain(e, carry):
            for _ in range(1 + NUM_NEG):
                pltpu.make_async_copy(eu_hbm.at[0], urows.at[0], sem).wait()
            return carry

        lax.fori_loop(0, CHUNK, drain, 0)

        def elem(e, carry, j=j):
            u = [urows[e, pl.ds(16 * t, 16)] for t in range(4)]
            off = (j * CHUNK + e) * PWN
            for k in range(NUM_NEG):
                r = e * NUM_NEG + k
                accn = nrows[r, pl.ds(0, 16)] * u[0]
                for t in range(1, 4):
                    accn += nrows[r, pl.ds(16 * t, 16)] * u[t]
                partials[pl.ds(off + k * 16, 16)] = -accn
            return carry

        lax.fori_loop(0, CHUNK, elem, 0)

        pltpu.sync_copy(urows, urows_hbm.at[pl.ds(base + j * CHUNK, CHUNK)])

    pltpu.sync_copy(partials, out_hbm.at[pl.ds(base * PWN, BPW * PWN)])


def _sc_v_body(cidx_hbm, ev_hbm, urows_hbm, out_hbm,
               cidx, vrows, urows, partials, sem):
    c = lax.axis_index("c")
    s = lax.axis_index("s")
    wid = s * NC + c
    base = wid * BPW

    pltpu.sync_copy(cidx_hbm.at[pl.ds(base, BPW)], cidx)

    for j in range(NCHUNK):
        pltpu.sync_copy(urows_hbm.at[pl.ds(base + j * CHUNK, CHUNK)], urows)

        def issue(g, carry, j=j):
            cv = cidx[pl.ds(j * CHUNK + g * 16, 16)]
            for l in range(16):
                pltpu.async_copy(ev_hbm.at[cv[l]], vrows.at[g * 16 + l], sem)
            return carry

        lax.fori_loop(0, CHUNK // 16, issue, 0)

        def drain(e, carry):
            pltpu.make_async_copy(ev_hbm.at[0], vrows.at[0], sem).wait()
            return carry

        lax.fori_loop(0, CHUNK, drain, 0)

        def elem(e, carry, j=j):
            accp = vrows[e, pl.ds(0, 16)] * urows[e, pl.ds(0, 16)]
            for t in range(1, 4):
                accp += (vrows[e, pl.ds(16 * t, 16)]
                         * urows[e, pl.ds(16 * t, 16)])
            partials[pl.ds((j * CHUNK + e) * 16, 16)] = accp
            return carry

        lax.fori_loop(0, CHUNK, elem, 0)

    pltpu.sync_copy(partials, out_hbm.at[pl.ds(base * 16, BPW * 16)])


def _sc_u(context, negflat, embed_u):
    mesh = plsc.VectorSubcoreMesh(core_axis_name="c", subcore_axis_name="s")
    f = pl.kernel(
        _sc_u_body,
        out_type=(jax.ShapeDtypeStruct((BATCH * PWN,), jnp.float32),
                  jax.ShapeDtypeStruct((BATCH, EMBED), jnp.float32)),
        mesh=mesh,
        scratch_types=[
            pltpu.VMEM((BPW,), jnp.int32),
            pltpu.VMEM((BPW * NUM_NEG,), jnp.int32),
            pltpu.VMEM((CHUNK, EMBED), jnp.float32),
            pltpu.VMEM((CHUNK * NUM_NEG, EMBED), jnp.float32),
            pltpu.VMEM((BPW * PWN,), jnp.float32),
            pltpu.SemaphoreType.DMA,
        ],
    )
    return f(context, negflat, embed_u)


def _sc_v(center, embed_v, urows_all):
    mesh = plsc.VectorSubcoreMesh(core_axis_name="c", subcore_axis_name="s")
    f = pl.kernel(
        _sc_v_body,
        out_type=jax.ShapeDtypeStruct((BATCH * 16,), jnp.float32),
        mesh=mesh,
        scratch_types=[
            pltpu.VMEM((BPW,), jnp.int32),
            pltpu.VMEM((CHUNK, EMBED), jnp.float32),
            pltpu.VMEM((CHUNK, EMBED), jnp.float32),
            pltpu.VMEM((BPW * 16,), jnp.float32),
            pltpu.SemaphoreType.DMA,
        ],
    )
    return f(center, embed_v, urows_all)


def _loss_body(pn_ref, pp_ref, o_ref):
    g = lax.broadcasted_iota(jnp.int32, (128, 8), 0) // 16
    t = lax.broadcasted_iota(jnp.int32, (128, 8), 1)
    m = jnp.where(g == t, 1.0, 0.0).astype(jnp.float32)
    sn = jax.lax.dot_general(pn_ref[...], m, (((1,), (0,)), ((), ())),
                             preferred_element_type=jnp.float32)
    sp = jax.lax.dot_general(pp_ref[...], m, (((1,), (0,)), ((), ())),
                             preferred_element_type=jnp.float32)
    tot = (jnp.sum(jnp.log(jax.nn.sigmoid(sn)))
           + jnp.sum(jnp.log(jax.nn.sigmoid(sp))))
    o_ref[0, 0] = tot * (-1.0 / BATCH)


def kernel(center_word, context_word, negative_samples, embed_v, embed_u):
    center = center_word.astype(jnp.int32)
    context = context_word.astype(jnp.int32)
    negflat = negative_samples.astype(jnp.int32).reshape(-1)
    pneg, urows_all = _sc_u(context, negflat, embed_u)
    ppos = _sc_v(center, embed_v, urows_all)
    pn2d = pneg.reshape(BATCH * PWN // 128, 128)
    pp2d = ppos.reshape(BATCH * 16 // 128, 128)
    loss = pl.pallas_call(
        _loss_body,
        out_shape=jax.ShapeDtypeStruct((1, 1), jnp.float32),
        out_specs=pl.BlockSpec(memory_space=pltpu.SMEM),
    )(pn2d, pp2d)
    return loss[0, 0]
